# Initial kernel scaffold; baseline (speedup 1.0000x reference)
import jax
import jax.numpy as jnp
from jax.experimental import pallas as pl


def kernel(embedding_mat, node_idxs, pos_idxs, neg_idxs):
    raise NotImplementedError("write your pallas kernel here")


# speedup vs baseline: 1.0091x; 1.0091x over previous
import functools

import jax
import jax.numpy as jnp
from jax import lax
from jax.experimental import pallas as pl
from jax.experimental.pallas import tpu as pltpu
from jax.experimental.pallas import tpu_sc as plsc

N_NODES = 100000
D = 128
P = 200000
NEG_NUM = 20
NEG_PAD = 32
Q = 10.0

NW = 32
W = 6272
PPAD = NW * W
C = 128
NCHUNK = W // C
ROWS2D = PPAD // 128


def _sc_scores_body(table, node_h, pos_h, neg_h, out_pos, out_neg,
                    idx_n, idx_p, rows_n, rows_p, nidx_v, nrows_v, ns_smem,
                    sc_pos, sc_neg, sem1, sem2):
    wid = lax.axis_index("s") * 2 + lax.axis_index("c")
    base = wid * W

    pltpu.sync_copy(neg_h, nidx_v)
    pltpu.async_copy(table.at[nidx_v], nrows_v, sem1).wait()
    for k in range(D // 16):
        acc = nrows_v[0, pl.ds(k * 16, 16)]
        for j in range(1, NEG_NUM):
            acc = acc + nrows_v[j, pl.ds(k * 16, 16)]
        for j in range(16):
            ns_smem[k * 16 + j] = acc[j]

    iota16 = lax.iota(jnp.int32, 16)

    def chunk_body(c, carry):
        off = base + c * C
        pltpu.sync_copy(node_h.at[pl.ds(off, C)], idx_n)
        pltpu.sync_copy(pos_h.at[pl.ds(off, C)], idx_p)
        cp1 = pltpu.async_copy(table.at[idx_n], rows_n, sem1)
        cp2 = pltpu.async_copy(table.at[idx_p], rows_p, sem2)
        cp1.wait()
        cp2.wait()
        for g in range(C // 16):
            rowv = iota16 + (g * 16)
            zf = jnp.zeros((16,), jnp.float32)

            def dbody(d, carry):
                colv, accp, accn = carry
                a = plsc.load_gather(rows_n, [rowv, colv])
                b = plsc.load_gather(rows_p, [rowv, colv])
                nsd = ns_smem[d]
                accp = accp + a * b
                accn = accn + a * nsd
                return (colv + 1, accp, accn)

            _, accp, accn = lax.fori_loop(
                0, D, dbody, (jnp.zeros((16,), jnp.int32), zf, zf),
                unroll=8)
            sc_pos[pl.ds(g * 16, 16)] = accp
            sc_neg[pl.ds(g * 16, 16)] = accn
        pltpu.sync_copy(sc_pos, out_pos.at[pl.ds(off, C)])
        pltpu.sync_copy(sc_neg, out_neg.at[pl.ds(off, C)])
        return carry

    lax.fori_loop(0, NCHUNK, chunk_body, 0)


_sc_scores = functools.partial(
    pl.kernel,
    out_type=(jax.ShapeDtypeStruct((PPAD,), jnp.float32),
              jax.ShapeDtypeStruct((PPAD,), jnp.float32)),
    mesh=plsc.VectorSubcoreMesh(core_axis_name="c", subcore_axis_name="s"),
    compiler_params=pltpu.CompilerParams(needs_layout_passes=False),
    scratch_types=[
        pltpu.VMEM((C,), jnp.int32),
        pltpu.VMEM((C,), jnp.int32),
        pltpu.VMEM((C, D), jnp.float32),
        pltpu.VMEM((C, D), jnp.float32),
        pltpu.VMEM((NEG_PAD,), jnp.int32),
        pltpu.VMEM((NEG_PAD, D), jnp.float32),
        pltpu.SMEM((D,), jnp.float32),
        pltpu.VMEM((C,), jnp.float32),
        pltpu.VMEM((C,), jnp.float32),
        pltpu.SemaphoreType.DMA,
        pltpu.SemaphoreType.DMA,
    ],
)(_sc_scores_body)


def _tc_loss_body(p_ref, n_ref, o_ref):
    x = p_ref[...]
    y = n_ref[...]
    rows = lax.broadcasted_iota(jnp.int32, (ROWS2D, 128), 0)
    cols = lax.broadcasted_iota(jnp.int32, (ROWS2D, 128), 1)
    valid = (rows * 128 + cols) < P

    def sp_neg(t):
        return jnp.maximum(-t, 0.0) + jnp.log1p(jnp.exp(-jnp.abs(t)))

    lp = jnp.where(valid, sp_neg(x), 0.0)
    ln = jnp.where(valid, sp_neg(y), 0.0)
    o_ref[0, 0] = jnp.sum(lp) / P + Q * (jnp.sum(ln) / P)


def _tc_loss(sp, sn):
    return pl.pallas_call(
        _tc_loss_body,
        out_shape=jax.ShapeDtypeStruct((1, 1), jnp.float32),
        out_specs=pl.BlockSpec(memory_space=pltpu.SMEM),
    )(sp, sn)


def kernel(embedding_mat, node_idxs, pos_idxs, neg_idxs):
    pad = jnp.zeros((PPAD - P,), jnp.int32)
    node_pad = jnp.concatenate([node_idxs, pad])
    pos_pad = jnp.concatenate([pos_idxs, pad])
    neg_pad = jnp.concatenate(
        [neg_idxs, jnp.zeros((NEG_PAD - NEG_NUM,), jnp.int32)])
    sp, sn = _sc_scores(embedding_mat, node_pad, pos_pad, neg_pad)
    loss = _tc_loss(sp.reshape(ROWS2D, 128), sn.reshape(ROWS2D, 128))
    return loss.reshape(1)


# speedup vs baseline: 4.2894x; 4.2507x over previous
import functools

import jax
import jax.numpy as jnp
from jax import lax
from jax.experimental import pallas as pl
from jax.experimental.pallas import tpu as pltpu
from jax.experimental.pallas import tpu_sc as plsc

N_NODES = 100000
D = 128
P = 200000
NEG_NUM = 20
NEG_PAD = 32
Q = 10.0

NW = 32
W = 6272
PPAD = NW * W
C = 112
NCHUNK = W // C
ROWS2D = PPAD // 128


def _sc_scores_body(table, node_h, pos_h, neg_h, out_pos, out_neg,
                    idx_n, idx_p, rows_n0, rows_p0, rows_n1, rows_p1,
                    nidx_v, nrows_v, ns_smem, sc_pos, sc_neg,
                    semn0, semp0, semn1, semp1, sem):
    wid = lax.axis_index("s") * 2 + lax.axis_index("c")
    base = wid * W

    cpn = pltpu.async_copy(node_h.at[pl.ds(base, W)], idx_n, semn0)
    cpp = pltpu.async_copy(pos_h.at[pl.ds(base, W)], idx_p, semp0)

    pltpu.sync_copy(neg_h, nidx_v)
    pltpu.async_copy(table.at[nidx_v], nrows_v, sem).wait()
    for k in range(D // 16):
        acc = nrows_v[0, pl.ds(k * 16, 16)]
        for j in range(1, NEG_NUM):
            acc = acc + nrows_v[j, pl.ds(k * 16, 16)]
        for j in range(16):
            ns_smem[k * 16 + j] = acc[j]
    cpn.wait()
    cpp.wait()

    rows_n = (rows_n0, rows_n1)
    rows_p = (rows_p0, rows_p1)
    semn = (semn0, semn1)
    semp = (semp0, semp1)

    def issue(c, b):
        off = c * C
        pltpu.async_copy(table.at[idx_n.at[pl.ds(off, C)]], rows_n[b],
                         semn[b])
        pltpu.async_copy(table.at[idx_p.at[pl.ds(off, C)]], rows_p[b],
                         semp[b])

    issue(0, 0)
    issue(1, 1)

    iota16 = lax.iota(jnp.int32, 16)

    def compute(c, b):
        pltpu.make_async_copy(table.at[idx_n.at[pl.ds(0, C)]], rows_n[b],
                              semn[b]).wait()
        pltpu.make_async_copy(table.at[idx_p.at[pl.ds(0, C)]], rows_p[b],
                              semp[b]).wait()
        rn, rp = rows_n[b], rows_p[b]
        for g in range(C // 16):
            rowv = iota16 + (g * 16)
            zf = jnp.zeros((16,), jnp.float32)

            def dbody(d, carry):
                colv, accp, accn = carry
                a = plsc.load_gather(rn, [rowv, colv])
                bb = plsc.load_gather(rp, [rowv, colv])
                nsd = ns_smem[d]
                accp = accp + a * bb
                accn = accn + a * nsd
                return (colv + 1, accp, accn)

            _, accp, accn = lax.fori_loop(
                0, D, dbody, (jnp.zeros((16,), jnp.int32), zf, zf),
                unroll=8)
            sc_pos[pl.ds(c * C + g * 16, 16)] = accp
            sc_neg[pl.ds(c * C + g * 16, 16)] = accn

    def pair_body(i2, carry):
        for b in range(2):
            c = i2 * 2 + b
            compute(c, b)

            @pl.when(c + 2 < NCHUNK)
            def _():
                issue(c + 2, b)
        return carry

    lax.fori_loop(0, NCHUNK // 2, pair_body, 0)

    pltpu.sync_copy(sc_pos, out_pos.at[pl.ds(base, W)])
    pltpu.sync_copy(sc_neg, out_neg.at[pl.ds(base, W)])


_sc_scores = functools.partial(
    pl.kernel,
    out_type=(jax.ShapeDtypeStruct((PPAD,), jnp.float32),
              jax.ShapeDtypeStruct((PPAD,), jnp.float32)),
    mesh=plsc.VectorSubcoreMesh(core_axis_name="c", subcore_axis_name="s"),
    compiler_params=pltpu.CompilerParams(needs_layout_passes=False),
    scratch_types=[
        pltpu.VMEM((W,), jnp.int32),
        pltpu.VMEM((W,), jnp.int32),
        pltpu.VMEM((C, D), jnp.float32),
        pltpu.VMEM((C, D), jnp.float32),
        pltpu.VMEM((C, D), jnp.float32),
        pltpu.VMEM((C, D), jnp.float32),
        pltpu.VMEM((NEG_PAD,), jnp.int32),
        pltpu.VMEM((NEG_PAD, D), jnp.float32),
        pltpu.SMEM((D,), jnp.float32),
        pltpu.VMEM((W,), jnp.float32),
        pltpu.VMEM((W,), jnp.float32),
        pltpu.SemaphoreType.DMA,
        pltpu.SemaphoreType.DMA,
        pltpu.SemaphoreType.DMA,
        pltpu.SemaphoreType.DMA,
        pltpu.SemaphoreType.DMA,
    ],
)(_sc_scores_body)


def _tc_loss_body(p_ref, n_ref, o_ref):
    x = p_ref[...]
    y = n_ref[...]
    rows = lax.broadcasted_iota(jnp.int32, (ROWS2D, 128), 0)
    cols = lax.broadcasted_iota(jnp.int32, (ROWS2D, 128), 1)
    valid = (rows * 128 + cols) < P

    def sp_neg(t):
        return jnp.maximum(-t, 0.0) + jnp.log1p(jnp.exp(-jnp.abs(t)))

    lp = jnp.where(valid, sp_neg(x), 0.0)
    ln = jnp.where(valid, sp_neg(y), 0.0)
    o_ref[0, 0] = jnp.sum(lp) / P + Q * (jnp.sum(ln) / P)


def _tc_loss(sp, sn):
    return pl.pallas_call(
        _tc_loss_body,
        out_shape=jax.ShapeDtypeStruct((1, 1), jnp.float32),
        out_specs=pl.BlockSpec(memory_space=pltpu.SMEM),
    )(sp, sn)


def kernel(embedding_mat, node_idxs, pos_idxs, neg_idxs):
    pad = jnp.zeros((PPAD - P,), jnp.int32)
    node_pad = jnp.concatenate([node_idxs, pad])
    pos_pad = jnp.concatenate([pos_idxs, pad])
    neg_pad = jnp.concatenate(
        [neg_idxs, jnp.zeros((NEG_PAD - NEG_NUM,), jnp.int32)])
    sp, sn = _sc_scores(embedding_mat, node_pad, pos_pad, neg_pad)
    loss = _tc_loss(sp.reshape(ROWS2D, 128), sn.reshape(ROWS2D, 128))
    return loss.reshape(1)
